# baseline (speedup 1.0000x reference)
import functools
import jax
import jax.numpy as jnp
from jax import lax
from jax.experimental import pallas as pl
from jax.experimental.pallas import tpu as pltpu
from jax.experimental.pallas import tpu_sc as plsc

_MIN_DIST = 4
_PRIMES = (2.0, 3.0, 5.0, 7.0)
_PAIR_PRODUCTS = (14.0, 15.0, 35.0)


def _prep_body(conr_ref, cont_ref, seqr_ref, seqc_ref, con2_ref):
    L = conr_ref.shape[0]
    ii = lax.broadcasted_iota(jnp.int32, (L, L), 0)
    jj = lax.broadcasted_iota(jnp.int32, (L, L), 1)
    sym = (conr_ref[...] + cont_ref[...]) * 0.5
    band = (jj - ii >= _MIN_DIST) | (ii - jj >= _MIN_DIST)
    sr = seqr_ref[...]
    m4r = jnp.max(sr, axis=0, keepdims=True)
    pr = jnp.full((1, L), _PRIMES[3], jnp.float32)
    pr = jnp.where(sr[2:3, :] == m4r, _PRIMES[2], pr)
    pr = jnp.where(sr[1:2, :] == m4r, _PRIMES[1], pr)
    pr = jnp.where(sr[0:1, :] == m4r, _PRIMES[0], pr)
    sc = seqc_ref[...]
    m4c = jnp.max(sc, axis=1, keepdims=True)
    pc = jnp.full((L, 1), _PRIMES[3], jnp.float32)
    pc = jnp.where(sc[:, 2:3] == m4c, _PRIMES[2], pc)
    pc = jnp.where(sc[:, 1:2] == m4c, _PRIMES[1], pc)
    pc = jnp.where(sc[:, 0:1] == m4c, _PRIMES[0], pc)
    pp = pc * pr
    pm = (pp == _PAIR_PRODUCTS[0]) | (pp == _PAIR_PRODUCTS[1]) \
        | (pp == _PAIR_PRODUCTS[2])
    pm = pm | (m4r < 1.0) | (m4c < 1.0)
    con2_ref[...] = jnp.where(band & pm, sym, 0.0)


def _prep(con2d, con2dT, seqr, seqc):
    L = con2d.shape[0]
    return pl.pallas_call(
        _prep_body,
        out_shape=jax.ShapeDtypeStruct((L, L), jnp.float32),
    )(con2d, con2dT, seqr, seqc)


@functools.lru_cache(maxsize=2)
def _make_sc_match(L):
    R = L // 16
    NCH = L // 16
    H = R // 2

    mesh = plsc.VectorSubcoreMesh(core_axis_name="c", subcore_axis_name="s")

    @functools.partial(
        pl.kernel,
        mesh=mesh,
        compiler_params=pltpu.CompilerParams(
            needs_layout_passes=False, use_tc_tiling_on_sc=True),
        out_type=jax.ShapeDtypeStruct((L,), jnp.int32),
        scratch_types=[
            pltpu.VMEM((R, L), jnp.float32),
            pltpu.VMEM((L,), jnp.float32),
            pltpu.VMEM((L,), jnp.int32),
            pltpu.VMEM((L,), jnp.float32),
            pltpu.VMEM((R,), jnp.int32),
            pltpu.VMEM((R,), jnp.float32),
            pltpu.VMEM((R,), jnp.int32),
            pltpu.VMEM_SHARED((2, L), jnp.int32),
            pltpu.VMEM_SHARED((2, L), jnp.float32),
            pltpu.SMEM((4,), jnp.int32),
            pltpu.SMEM((L // 16,), jnp.int32),
        ],
    )
    def sc_match(con2_hbm, out_hbm, rows_v, free_v, a_full, m_full, a_own,
                 m_own, mcol_v, a_sh, m_sh, st, rs):
        c = lax.axis_index("c")
        s = lax.axis_index("s")
        row0 = s * R
        iota16 = lax.broadcasted_iota(jnp.int32, (16,), 0)
        lane0 = iota16 == 0
        ones16 = jnp.ones((16,), jnp.float32)
        zeros16 = jnp.zeros((16,), jnp.float32)

        pltpu.sync_copy(con2_hbm.at[pl.ds(row0, R), :], rows_v)

        for k in range(NCH):
            free_v[pl.ds(k * 16, 16)] = ones16
        for k in range(R // 16):
            mcol_v[pl.ds(k * 16, 16)] = jnp.full((16,), -1, jnp.int32)

        st[0] = jnp.int32(L + 1)
        st[1] = jnp.int32(1)

        def _finish_row(r, bv, bc):
            sk, _ = plsc.sort_key_val(bv, bc, descending=True)
            mx = sk[0]
            cand = jnp.where(bv == mx, bc, L)
            ck, _ = plsc.sort_key_val(cand, cand)
            am = ck[0]
            idx16 = jnp.full((16,), r, jnp.int32)
            plsc.store_scatter(m_own, [idx16],
                               jnp.full((16,), mx, jnp.float32),
                               mask=lane0)
            plsc.store_scatter(a_own, [idx16],
                               jnp.full((16,), am, jnp.int32),
                               mask=lane0)

        def _rest_of_round(parity):
            pltpu.sync_copy(a_own, a_sh.at[parity, pl.ds(row0, R)])
            pltpu.sync_copy(m_own, m_sh.at[parity, pl.ds(row0, R)])
            plsc.subcore_barrier()
            pltpu.sync_copy(a_sh.at[parity], a_full)
            pltpu.sync_copy(m_sh.at[parity], m_full)

            for k in range(R // 16):
                g16 = iota16 + (row0 + k * 16)
                a16 = a_own[pl.ds(k * 16, 16)]
                m16 = m_own[pl.ds(k * 16, 16)]
                f16 = plsc.load_gather(free_v, [g16])
                partner = plsc.load_gather(a_full, [a16])
                mut = (partner == g16) & (m16 > 0.0) & (f16 > 0.0)
                mcol_v[pl.ds(k * 16, 16)] = jnp.where(
                    mut, a16, mcol_v[pl.ds(k * 16, 16)])

            for k in range(NCH):
                g16 = iota16 + (k * 16)
                a16 = a_full[pl.ds(k * 16, 16)]
                m16 = m_full[pl.ds(k * 16, 16)]
                f16 = free_v[pl.ds(k * 16, 16)]
                partner = plsc.load_gather(a_full, [a16])
                mut = (partner == g16) & (m16 > 0.0) & (f16 > 0.0)
                free_v[pl.ds(k * 16, 16)] = jnp.where(mut, 0.0, f16)

            for k in range(R // 16):
                g16 = iota16 + (row0 + k * 16)
                a16 = a_own[pl.ds(k * 16, 16)]
                m16 = m_own[pl.ds(k * 16, 16)]
                f16 = plsc.load_gather(free_v, [g16])
                pf16 = plsc.load_gather(free_v, [a16])
                need = ((f16 > 0.0) & (m16 > 0.0) & (pf16 == 0.0))
                need_i = need.astype(jnp.int32)
                for lane in range(16):
                    rs[k * 16 + lane] = need_i[lane]

            cntv = jnp.zeros((16,), jnp.int32)
            for ch in range(NCH):
                cntv = cntv + plsc.all_reduce_population_count(
                    free_v[pl.ds(ch * 16, 16)] > 0.0)
            cnt = cntv[0]
            go = jnp.where((cnt < st[0]) & (cnt > 0), 1, 0)
            st[0] = cnt
            st[1] = go

        def round_body(i, _):
            @pl.when(st[1] > 0)
            def _round():
                def row_body(r, _):
                    @pl.when(rs[r] > 0)
                    def _scan():
                        bv = jnp.full((16,), -1.0, jnp.float32)
                        bc = jnp.zeros((16,), jnp.int32)
                        for ch in range(NCH):
                            v = rows_v[r, pl.ds(ch * 16, 16)] \
                                * free_v[pl.ds(ch * 16, 16)]
                            gt = v > bv
                            bv = jnp.where(gt, v, bv)
                            bc = jnp.where(gt, iota16 + (ch * 16), bc)
                        _finish_row(r, bv, bc)
                    return 0
                lax.fori_loop(0, R, row_body, 0)
                _rest_of_round((i + 1) % 2)
            return 0

        def row1_body(r, _):
            bv = jnp.full((16,), -1.0, jnp.float32)
            bc = jnp.zeros((16,), jnp.int32)
            for ch in range(NCH):
                v = rows_v[r, pl.ds(ch * 16, 16)]
                gt = v > bv
                bv = jnp.where(gt, v, bv)
                bc = jnp.where(gt, iota16 + (ch * 16), bc)
            _finish_row(r, bv, bc)
            return 0
        lax.fori_loop(0, R, row1_body, 0)
        _rest_of_round(0)

        lax.fori_loop(0, L // 2 + 2, round_body, 0)

        @pl.when(c == 0)
        def _():
            pltpu.sync_copy(mcol_v, out_hbm.at[pl.ds(row0, R)])

    return sc_match


def _mask_body(con2_ref, mcol_ref, out_ref):
    L = con2_ref.shape[0]
    jj = lax.broadcasted_iota(jnp.int32, (L, L), 1)
    out_ref[...] = jnp.where(jj == mcol_ref[...], con2_ref[...], 0.0)


def _mask(con2, mcol_col):
    L = con2.shape[0]
    return pl.pallas_call(
        _mask_body,
        out_shape=jax.ShapeDtypeStruct((L, L), jnp.float32),
    )(con2, mcol_col)


def kernel(con, feat):
    shape = con.shape
    L = shape[-1]
    con2d = con.reshape(L, L)
    con2dT = jnp.swapaxes(con2d, 0, 1)
    seqr = feat.reshape(feat.shape[1], L, L)[:4, :, 0]
    seqc = jnp.swapaxes(seqr, 0, 1)
    con2 = _prep(con2d, con2dT, seqr, seqc)
    mcol = _make_sc_match(L)(con2)
    out = _mask(con2, mcol.reshape(L, 1))
    return out.reshape(shape)


# speedup vs baseline: 2.5573x; 2.5573x over previous
import functools
import jax
import jax.numpy as jnp
from jax import lax
from jax.experimental import pallas as pl
from jax.experimental.pallas import tpu as pltpu
from jax.experimental.pallas import tpu_sc as plsc

_MIN_DIST = 4
_PRIMES = (2.0, 3.0, 5.0, 7.0)
_PAIR_PRODUCTS = (14.0, 15.0, 35.0)


def _prep_body(conr_ref, seqr_ref, seqc_ref, con2_ref):
    L = conr_ref.shape[0]
    ii = lax.broadcasted_iota(jnp.int32, (L, L), 0)
    jj = lax.broadcasted_iota(jnp.int32, (L, L), 1)
    cr = conr_ref[...]
    sym = (cr + cr.T) * 0.5
    band = (jj - ii >= _MIN_DIST) | (ii - jj >= _MIN_DIST)
    sr = seqr_ref[...]
    m4r = jnp.max(sr, axis=0, keepdims=True)
    pr = jnp.full((1, L), _PRIMES[3], jnp.float32)
    pr = jnp.where(sr[2:3, :] == m4r, _PRIMES[2], pr)
    pr = jnp.where(sr[1:2, :] == m4r, _PRIMES[1], pr)
    pr = jnp.where(sr[0:1, :] == m4r, _PRIMES[0], pr)
    sc = seqc_ref[...]
    m4c = jnp.max(sc, axis=1, keepdims=True)
    pc = jnp.full((L, 1), _PRIMES[3], jnp.float32)
    pc = jnp.where(sc[:, 2:3] == m4c, _PRIMES[2], pc)
    pc = jnp.where(sc[:, 1:2] == m4c, _PRIMES[1], pc)
    pc = jnp.where(sc[:, 0:1] == m4c, _PRIMES[0], pc)
    pp = pc * pr
    pm = (pp == _PAIR_PRODUCTS[0]) | (pp == _PAIR_PRODUCTS[1]) \
        | (pp == _PAIR_PRODUCTS[2])
    pm = pm | (m4r < 1.0) | (m4c < 1.0)
    con2_ref[...] = jnp.where(band & pm, sym, 0.0)


def _prep(con2d, seqr, seqc):
    L = con2d.shape[0]
    return pl.pallas_call(
        _prep_body,
        out_shape=jax.ShapeDtypeStruct((L, L), jnp.float32),
    )(con2d, seqr, seqc)


@functools.lru_cache(maxsize=2)
def _make_sc_match(L):
    R = L // 16
    NCH = L // 16
    H = R // 2

    mesh = plsc.VectorSubcoreMesh(core_axis_name="c", subcore_axis_name="s")

    @functools.partial(
        pl.kernel,
        mesh=mesh,
        compiler_params=pltpu.CompilerParams(needs_layout_passes=False, use_tc_tiling_on_sc=True),
        out_type=jax.ShapeDtypeStruct((L, L), jnp.float32),
        scratch_types=[
            pltpu.VMEM((R, L), jnp.float32),
            pltpu.VMEM((L,), jnp.float32),
            pltpu.VMEM((R,), jnp.float32),
            pltpu.VMEM((L,), jnp.int32),
            pltpu.VMEM((R,), jnp.int32),
            pltpu.VMEM((R,), jnp.float32),
            pltpu.VMEM((R,), jnp.int32),
            pltpu.VMEM((R,), jnp.float32),
            pltpu.VMEM_SHARED((L,), jnp.int32),
            pltpu.VMEM_SHARED((L,), jnp.float32),
            pltpu.SMEM((4,), jnp.int32),
            pltpu.SMEM((L // 16,), jnp.int32),
        ],
    )
    def sc_match(con2_hbm, out_hbm, rows_v, free_v, free_own, a_full, a_own,
                 m_own, mcol_v, mval_v, a_sh, free_sh, st, rs):
        c = lax.axis_index("c")
        s = lax.axis_index("s")
        row0 = s * R
        iota16 = lax.broadcasted_iota(jnp.int32, (16,), 0)
        lane0 = iota16 == 0
        ones16 = jnp.ones((16,), jnp.float32)
        zeros16 = jnp.zeros((16,), jnp.float32)

        pltpu.sync_copy(con2_hbm.at[pl.ds(row0, R), :], rows_v)

        for k in range(NCH):
            free_v[pl.ds(k * 16, 16)] = ones16
        for k in range(R // 16):
            free_own[pl.ds(k * 16, 16)] = ones16
            mcol_v[pl.ds(k * 16, 16)] = jnp.full((16,), -1, jnp.int32)
            mval_v[pl.ds(k * 16, 16)] = zeros16

        st[0] = jnp.int32(L + 1)
        st[1] = jnp.int32(1)
        for r in range(R):
            rs[r] = jnp.int32(1)

        def _finish_row(r, bv, bc):
            sk, _ = plsc.sort_key_val(bv, bc, descending=True)
            mx = sk[0]
            cand = jnp.where(bv == mx, bc, L)
            ck, _ = plsc.sort_key_val(cand, cand)
            am = ck[0]
            idx16 = jnp.full((16,), r, jnp.int32)
            plsc.store_scatter(m_own, [idx16],
                               jnp.full((16,), mx, jnp.float32),
                               mask=lane0)
            plsc.store_scatter(a_own, [idx16],
                               jnp.full((16,), am, jnp.int32),
                               mask=lane0)

        def round_body(i, _):
            @pl.when(st[1] > 0)
            def _round():
                def row_body(r, _):
                    @pl.when(rs[r] > 0)
                    def _scan():
                        bv = jnp.full((16,), -1.0, jnp.float32)
                        bc = jnp.zeros((16,), jnp.int32)
                        for ch in range(NCH):
                            v = rows_v[r, pl.ds(ch * 16, 16)] \
                                * free_v[pl.ds(ch * 16, 16)]
                            gt = v > bv
                            bv = jnp.where(gt, v, bv)
                            bc = jnp.where(gt, iota16 + (ch * 16), bc)
                        _finish_row(r, bv, bc)
                    return 0
                lax.fori_loop(0, R, row_body, 0)
                _rest_of_round()
            return 0

        def _rest_of_round():
            pltpu.sync_copy(a_own, a_sh.at[pl.ds(row0, R)])
            plsc.subcore_barrier()
            pltpu.sync_copy(a_sh, a_full)

            for k in range(R // 16):
                g16 = iota16 + (row0 + k * 16)
                a16 = a_own[pl.ds(k * 16, 16)]
                m16 = m_own[pl.ds(k * 16, 16)]
                f16 = free_own[pl.ds(k * 16, 16)]
                partner = plsc.load_gather(a_full, [a16])
                mut = (partner == g16) & (m16 > 0.0) & (f16 > 0.0)
                mcol_v[pl.ds(k * 16, 16)] = jnp.where(
                    mut, a16, mcol_v[pl.ds(k * 16, 16)])
                mval_v[pl.ds(k * 16, 16)] = jnp.where(
                    mut, m16, mval_v[pl.ds(k * 16, 16)])
                free_own[pl.ds(k * 16, 16)] = jnp.where(mut, 0.0, f16)

            pltpu.sync_copy(free_own, free_sh.at[pl.ds(row0, R)])
            plsc.subcore_barrier()
            pltpu.sync_copy(free_sh, free_v)

            for k in range(R // 16):
                a16 = a_own[pl.ds(k * 16, 16)]
                m16 = m_own[pl.ds(k * 16, 16)]
                f16 = free_own[pl.ds(k * 16, 16)]
                pf16 = plsc.load_gather(free_v, [a16])
                need = ((f16 > 0.0) & (m16 > 0.0) & (pf16 == 0.0))
                need_i = need.astype(jnp.int32)
                for lane in range(16):
                    rs[k * 16 + lane] = need_i[lane]

            cntv = jnp.zeros((16,), jnp.int32)
            for ch in range(NCH):
                cntv = cntv + plsc.all_reduce_population_count(
                    free_v[pl.ds(ch * 16, 16)] > 0.0)
            cnt = cntv[0]
            go = jnp.where((cnt < st[0]) & (cnt > 0), 1, 0)
            st[0] = cnt
            st[1] = go

        def row1_body(r, _):
            bv = jnp.full((16,), -1.0, jnp.float32)
            bc = jnp.zeros((16,), jnp.int32)
            for ch in range(NCH):
                v = rows_v[r, pl.ds(ch * 16, 16)]
                gt = v > bv
                bv = jnp.where(gt, v, bv)
                bc = jnp.where(gt, iota16 + (ch * 16), bc)
            _finish_row(r, bv, bc)
            return 0
        lax.fori_loop(0, R, row1_body, 0)
        _rest_of_round()

        lax.fori_loop(0, L // 2 + 2, round_body, 0)

        def write_half(lbase):
            def zrow(r, _):
                for ch in range(NCH):
                    rows_v[lbase + r, pl.ds(ch * 16, 16)] = zeros16
                return 0
            lax.fori_loop(0, H, zrow, 0)
            for k in range(H // 16):
                lrow16 = iota16 + (lbase + k * 16)
                mc16 = mcol_v[pl.ds(lbase + k * 16, 16)]
                mv16 = mval_v[pl.ds(lbase + k * 16, 16)]
                plsc.store_scatter(rows_v, [lrow16, mc16], mv16,
                                   mask=mc16 >= 0)
            pltpu.sync_copy(rows_v.at[pl.ds(lbase, H), :],
                            out_hbm.at[pl.ds(row0 + lbase, H), :])

        @pl.when(c == 0)
        def _():
            write_half(0)

        @pl.when(c == 1)
        def _():
            write_half(H)

    return sc_match


def kernel(con, feat):
    shape = con.shape
    L = shape[-1]
    con2d = con.reshape(L, L)
    feat3 = feat.reshape(feat.shape[1], L, L)[:4]
    e0 = (jnp.arange(L) == 0).astype(jnp.float32)
    seqr = jnp.einsum("cij,j->ci", feat3, e0,
                      preferred_element_type=jnp.float32)
    seqc = jnp.swapaxes(seqr, 0, 1)
    con2 = _prep(con2d, seqr, seqc)
    out = _make_sc_match(L)(con2)
    return out.reshape(shape)
